# baseline (speedup 1.0000x reference)
import functools

import jax
import jax.numpy as jnp
from jax import lax
from jax.experimental import pallas as pl
from jax.experimental.pallas import tpu as pltpu
from jax.experimental.pallas import tpu_sc as plsc

_C = 85
_S = 256
_NRV = _S // 16
_NCHUNK = 4096 // _S
_PAIRS = 96
_ITEMS = _PAIRS * _NCHUNK
_NW = 32
_PAIRS_PER_W = _PAIRS // _NW
_STRIDE = 8.0
_OUT_ITEM = _S * _C
_IN_ITEM = _C * _S


def _sc_body(x_hbm, out_hbm, in_v, out_v, sem):
    wid = lax.axis_index("s") * 2 + lax.axis_index("c")
    lane = lax.iota(jnp.int32, 16)
    lane85 = lane * _C

    def pair_body(pi, carry):
        pair = wid * _PAIRS_PER_W + pi
        a = lax.rem(pair, 3)
        aw = jnp.where(a == 0, 10.0, jnp.where(a == 1, 16.0, 33.0))
        ah = jnp.where(a == 0, 13.0, jnp.where(a == 1, 30.0, 23.0))
        pair_base = pair * _C * 4096

        def chunk_body(ci, carry2):
            s0 = ci * _S

            def fire(c, carry3):
                pltpu.async_copy(
                    x_hbm.at[pl.ds(pair_base + c * 4096 + s0, _S)],
                    in_v.at[pl.ds(c * _S, _S)], sem)
                return carry3

            lax.fori_loop(0, _C, fire, 0, unroll=False)
            pltpu.make_async_copy(
                x_hbm.at[pl.ds(0, _IN_ITEM)], in_v, sem).wait()

            def ch_body(c, carry3):
                for rv in range(_NRV):
                    v = in_v[pl.ds(c * _S + rv * 16, 16)]
                    e = jnp.exp(v)
                    sig = e / (1.0 + e)
                    idx = lane85 + (rv * 16 * _C + c)
                    plsc.store_scatter(out_v, [idx], sig)
                return carry3

            lax.fori_loop(4, _C, ch_body, 0, unroll=False)

            for rv in range(_NRV):
                r_global = lane + (s0 + rv * 16)
                gx = (r_global & 63).astype(jnp.float32)
                gy = (r_global >> 6).astype(jnp.float32)
                base_idx = lane85 + rv * 16 * _C

                v0 = in_v[pl.ds(rv * 16, 16)]
                e0 = jnp.exp(v0)
                plsc.store_scatter(out_v, [base_idx],
                                   (e0 / (1.0 + e0) + gx) * _STRIDE)
                v1 = in_v[pl.ds(_S + rv * 16, 16)]
                e1 = jnp.exp(v1)
                plsc.store_scatter(out_v, [base_idx + 1],
                                   (e1 / (1.0 + e1) + gy) * _STRIDE)
                v2 = in_v[pl.ds(2 * _S + rv * 16, 16)]
                plsc.store_scatter(out_v, [base_idx + 2], jnp.exp(v2) * aw)
                v3 = in_v[pl.ds(3 * _S + rv * 16, 16)]
                plsc.store_scatter(out_v, [base_idx + 3], jnp.exp(v3) * ah)

            pltpu.sync_copy(
                out_v,
                out_hbm.at[pl.ds((pair * _NCHUNK + ci) * _OUT_ITEM, _OUT_ITEM)])
            return carry2

        lax.fori_loop(0, _NCHUNK, chunk_body, 0, unroll=False)
        return carry

    lax.fori_loop(0, _PAIRS_PER_W, pair_body, 0, unroll=False)


_sc_call = functools.partial(
    pl.kernel,
    mesh=plsc.VectorSubcoreMesh(core_axis_name="c", subcore_axis_name="s"),
    out_type=jax.ShapeDtypeStruct((_ITEMS * _OUT_ITEM,), jnp.float32),
    scratch_types=[
        pltpu.VMEM((_IN_ITEM,), jnp.float32),
        pltpu.VMEM((_OUT_ITEM,), jnp.float32),
        pltpu.SemaphoreType.DMA,
    ],
    compiler_params=pltpu.CompilerParams(
        needs_layout_passes=False, use_tc_tiling_on_sc=True),
)(_sc_body)


@jax.jit
def kernel(x):
    B = x.shape[0]
    x1 = x.reshape(B * 3 * _C * 64 * 64)
    out = _sc_call(x1)
    return out.reshape(B, 3 * 64 * 64, _C)


# speedup vs baseline: 2.0325x; 2.0325x over previous
import functools

import jax
import jax.numpy as jnp
from jax import lax
from jax.experimental import pallas as pl
from jax.experimental.pallas import tpu as pltpu
from jax.experimental.pallas import tpu_sc as plsc

_C = 85
_S = 256
_NRV = _S // 16
_NCHUNK = 4096 // _S
_PAIRS = 96
_ITEMS = _PAIRS * _NCHUNK
_NW = 32
_PAIRS_PER_W = _PAIRS // _NW
_STRIDE = 8.0
_OUT_ITEM = _S * _C
_IN_ITEM = _C * _S
_NGEN = (_C - 4) * _NRV


def _sc_body(x_hbm, out_hbm, in_v, out_v, sem):
    wid = lax.axis_index("s") * 2 + lax.axis_index("c")
    lane = lax.iota(jnp.int32, 16)
    lane85 = lane * _C

    def pair_body(pi, carry):
        pair = wid * _PAIRS_PER_W + pi
        a = lax.rem(pair, 3)
        aw = jnp.where(a == 0, 10.0, jnp.where(a == 1, 16.0, 33.0))
        ah = jnp.where(a == 0, 13.0, jnp.where(a == 1, 30.0, 23.0))
        pair_base = pair * _C * 4096

        def chunk_body(ci, carry2):
            s0 = ci * _S

            def fire(c, carry3):
                pltpu.async_copy(
                    x_hbm.at[pl.ds(pair_base + c * 4096 + s0, _S)],
                    in_v.at[pl.ds(c * _S, _S)], sem)
                return carry3

            lax.fori_loop(0, _C, fire, 0, unroll=False)
            pltpu.make_async_copy(
                x_hbm.at[pl.ds(0, _IN_ITEM)], in_v, sem).wait()

            @plsc.parallel_loop(0, _NGEN, unroll=4)
            def _gen(t):
                v = in_v[pl.ds(4 * _S + t * 16, 16)]
                e = jnp.exp(v)
                sig = e / (1.0 + e)
                rv = t & (_NRV - 1)
                c = (t >> 4) + 4
                idx = lane85 + (rv * 16 * _C + c)
                plsc.store_scatter(out_v, [idx], sig)

            @plsc.parallel_loop(0, _NRV, unroll=2)
            def _box(rv):
                r_global = lane + (s0 + rv * 16)
                gx = (r_global & 63).astype(jnp.float32)
                gy = (r_global >> 6).astype(jnp.float32)
                base_idx = lane85 + rv * 16 * _C

                v0 = in_v[pl.ds(rv * 16, 16)]
                e0 = jnp.exp(v0)
                plsc.store_scatter(out_v, [base_idx],
                                   (e0 / (1.0 + e0) + gx) * _STRIDE)
                v1 = in_v[pl.ds(_S + rv * 16, 16)]
                e1 = jnp.exp(v1)
                plsc.store_scatter(out_v, [base_idx + 1],
                                   (e1 / (1.0 + e1) + gy) * _STRIDE)
                v2 = in_v[pl.ds(2 * _S + rv * 16, 16)]
                plsc.store_scatter(out_v, [base_idx + 2], jnp.exp(v2) * aw)
                v3 = in_v[pl.ds(3 * _S + rv * 16, 16)]
                plsc.store_scatter(out_v, [base_idx + 3], jnp.exp(v3) * ah)

            pltpu.sync_copy(
                out_v,
                out_hbm.at[pl.ds((pair * _NCHUNK + ci) * _OUT_ITEM, _OUT_ITEM)])
            return carry2

        lax.fori_loop(0, _NCHUNK, chunk_body, 0, unroll=False)
        return carry

    lax.fori_loop(0, _PAIRS_PER_W, pair_body, 0, unroll=False)


_sc_call = functools.partial(
    pl.kernel,
    mesh=plsc.VectorSubcoreMesh(core_axis_name="c", subcore_axis_name="s"),
    out_type=jax.ShapeDtypeStruct((_ITEMS * _OUT_ITEM,), jnp.float32),
    scratch_types=[
        pltpu.VMEM((_IN_ITEM,), jnp.float32),
        pltpu.VMEM((_OUT_ITEM,), jnp.float32),
        pltpu.SemaphoreType.DMA,
    ],
    compiler_params=pltpu.CompilerParams(
        needs_layout_passes=False, use_tc_tiling_on_sc=True),
)(_sc_body)


@jax.jit
def kernel(x):
    B = x.shape[0]
    x1 = x.reshape(B * 3 * _C * 64 * 64)
    out = _sc_call(x1)
    return out.reshape(B, 3 * 64 * 64, _C)


# speedup vs baseline: 2.2610x; 1.1124x over previous
import functools

import jax
import jax.numpy as jnp
from jax import lax
from jax.experimental import pallas as pl
from jax.experimental.pallas import tpu as pltpu
from jax.experimental.pallas import tpu_sc as plsc

_C = 85
_S = 256
_NRV = _S // 16
_NCHUNK = 4096 // _S
_PAIRS = 96
_ITEMS = _PAIRS * _NCHUNK
_NW = 32
_PAIRS_PER_W = _PAIRS // _NW
_ITEMS_PER_W = _ITEMS // _NW
_STRIDE = 8.0
_OUT_ITEM = _S * _C
_IN_ITEM = _C * _S
_NGEN = (_C - 4) * _NRV


def _sc_body(x_hbm, out_hbm, in_a, in_b, out_a, out_b,
             isem_a, isem_b, osem_a, osem_b):
    wid = lax.axis_index("s") * 2 + lax.axis_index("c")
    lane = lax.iota(jnp.int32, 16)
    lane85 = lane * _C

    def fire(t, in_ref, isem):
        base = (wid * 3 + (t >> 4)) * _C * 4096 + (t & (_NCHUNK - 1)) * _S

        def row(c, carry):
            pltpu.async_copy(
                x_hbm.at[pl.ds(base + c * 4096, _S)],
                in_ref.at[pl.ds(c * _S, _S)], isem)
            return carry

        lax.fori_loop(0, _C, row, 0, unroll=False)

    def wait_in(in_ref, isem):
        pltpu.make_async_copy(x_hbm.at[pl.ds(0, _IN_ITEM)], in_ref, isem).wait()

    def drain_out(out_ref, osem):
        pltpu.make_async_copy(x_hbm.at[pl.ds(0, _OUT_ITEM)], out_ref, osem).wait()

    def compute(t, in_ref, out_ref, osem):
        pi = t >> 4
        aw = jnp.where(pi == 0, 10.0, jnp.where(pi == 1, 16.0, 33.0))
        ah = jnp.where(pi == 0, 13.0, jnp.where(pi == 1, 30.0, 23.0))
        s0 = (t & (_NCHUNK - 1)) * _S

        @plsc.parallel_loop(0, _NGEN, unroll=4)
        def _gen(g):
            v = in_ref[pl.ds(4 * _S + g * 16, 16)]
            e = jnp.exp(v)
            sig = e / (1.0 + e)
            idx = lane85 + ((g & (_NRV - 1)) * 16 * _C + (g >> 4) + 4)
            plsc.store_scatter(out_ref, [idx], sig)

        @plsc.parallel_loop(0, _NRV, unroll=2)
        def _box(rv):
            r_global = lane + (s0 + rv * 16)
            gx = (r_global & 63).astype(jnp.float32)
            gy = (r_global >> 6).astype(jnp.float32)
            base_idx = lane85 + rv * 16 * _C

            v0 = in_ref[pl.ds(rv * 16, 16)]
            e0 = jnp.exp(v0)
            plsc.store_scatter(out_ref, [base_idx],
                               (e0 / (1.0 + e0) + gx) * _STRIDE)
            v1 = in_ref[pl.ds(_S + rv * 16, 16)]
            e1 = jnp.exp(v1)
            plsc.store_scatter(out_ref, [base_idx + 1],
                               (e1 / (1.0 + e1) + gy) * _STRIDE)
            v2 = in_ref[pl.ds(2 * _S + rv * 16, 16)]
            plsc.store_scatter(out_ref, [base_idx + 2], jnp.exp(v2) * aw)
            v3 = in_ref[pl.ds(3 * _S + rv * 16, 16)]
            plsc.store_scatter(out_ref, [base_idx + 3], jnp.exp(v3) * ah)

        out_base = ((wid * 3 + pi) * _NCHUNK + (t & (_NCHUNK - 1))) * _OUT_ITEM
        pltpu.async_copy(out_ref, out_hbm.at[pl.ds(out_base, _OUT_ITEM)], osem)

    fire(0, in_a, isem_a)
    fire(1, in_b, isem_b)

    def step(t2, carry):
        ta = t2 * 2
        tb = ta + 1

        @pl.when(t2 > 0)
        def _():
            drain_out(out_a, osem_a)

        wait_in(in_a, isem_a)
        compute(ta, in_a, out_a, osem_a)

        @pl.when(ta + 2 < _ITEMS_PER_W)
        def _():
            fire(ta + 2, in_a, isem_a)

        @pl.when(t2 > 0)
        def _():
            drain_out(out_b, osem_b)

        wait_in(in_b, isem_b)
        compute(tb, in_b, out_b, osem_b)

        @pl.when(tb + 2 < _ITEMS_PER_W)
        def _():
            fire(tb + 2, in_b, isem_b)

        return carry

    lax.fori_loop(0, _ITEMS_PER_W // 2, step, 0, unroll=False)
    drain_out(out_a, osem_a)
    drain_out(out_b, osem_b)


_sc_call = functools.partial(
    pl.kernel,
    mesh=plsc.VectorSubcoreMesh(core_axis_name="c", subcore_axis_name="s"),
    out_type=jax.ShapeDtypeStruct((_ITEMS * _OUT_ITEM,), jnp.float32),
    scratch_types=[
        pltpu.VMEM((_IN_ITEM,), jnp.float32),
        pltpu.VMEM((_IN_ITEM,), jnp.float32),
        pltpu.VMEM((_OUT_ITEM,), jnp.float32),
        pltpu.VMEM((_OUT_ITEM,), jnp.float32),
        pltpu.SemaphoreType.DMA,
        pltpu.SemaphoreType.DMA,
        pltpu.SemaphoreType.DMA,
        pltpu.SemaphoreType.DMA,
    ],
    compiler_params=pltpu.CompilerParams(
        needs_layout_passes=False, use_tc_tiling_on_sc=True),
)(_sc_body)


@jax.jit
def kernel(x):
    B = x.shape[0]
    x1 = x.reshape(B * 3 * _C * 64 * 64)
    out = _sc_call(x1)
    return out.reshape(B, 3 * 64 * 64, _C)


# speedup vs baseline: 2.3628x; 1.0450x over previous
import functools

import jax
import jax.numpy as jnp
from jax import lax
from jax.experimental import pallas as pl
from jax.experimental.pallas import tpu as pltpu
from jax.experimental.pallas import tpu_sc as plsc

_C = 85
_S = 256
_NRV = _S // 16
_NCHUNK = 4096 // _S
_PAIRS = 96
_ITEMS = _PAIRS * _NCHUNK
_NW = 32
_PAIRS_PER_W = _PAIRS // _NW
_ITEMS_PER_W = _ITEMS // _NW
_STRIDE = 8.0
_OUT_ITEM = _S * _C
_IN_ITEM = _C * _S
_NGEN = (_C - 4) * _NRV


def _sc_body(x_hbm, out_hbm, in_a, in_b, out_a, out_b,
             isem_a, isem_b, osem_a, osem_b):
    wid = lax.axis_index("s") * 2 + lax.axis_index("c")
    lane = lax.iota(jnp.int32, 16)
    lane85 = lane * _C

    def fire(t, in_ref, isem):
        base = (wid * 3 + (t >> 4)) * _C * 4096 + (t & (_NCHUNK - 1)) * _S

        def row(c, carry):
            pltpu.async_copy(
                x_hbm.at[pl.ds(base + c * 4096, _S)],
                in_ref.at[pl.ds(c * _S, _S)], isem)
            return carry

        lax.fori_loop(0, _C, row, 0, unroll=False)

    def wait_in(in_ref, isem):
        pltpu.make_async_copy(x_hbm.at[pl.ds(0, _IN_ITEM)], in_ref, isem).wait()

    def drain_out(out_ref, osem):
        pltpu.make_async_copy(x_hbm.at[pl.ds(0, _OUT_ITEM)], out_ref, osem).wait()

    def compute(t, in_ref, out_ref, osem):
        pi = t >> 4
        aw = jnp.where(pi == 0, 10.0, jnp.where(pi == 1, 16.0, 33.0))
        ah = jnp.where(pi == 0, 13.0, jnp.where(pi == 1, 30.0, 23.0))
        s0 = (t & (_NCHUNK - 1)) * _S

        @plsc.parallel_loop(0, _NGEN, unroll=8)
        def _gen(g):
            v = in_ref[pl.ds(4 * _S + g * 16, 16)]
            e = jnp.exp(v)
            sig = e / (1.0 + e)
            idx = lane85 + ((g & (_NRV - 1)) * 16 * _C + (g >> 4) + 4)
            plsc.store_scatter(out_ref, [idx], sig)

        @plsc.parallel_loop(0, _NRV, unroll=4)
        def _box(rv):
            r_global = lane + (s0 + rv * 16)
            gx = (r_global & 63).astype(jnp.float32)
            gy = (r_global >> 6).astype(jnp.float32)
            base_idx = lane85 + rv * 16 * _C

            v0 = in_ref[pl.ds(rv * 16, 16)]
            e0 = jnp.exp(v0)
            plsc.store_scatter(out_ref, [base_idx],
                               (e0 / (1.0 + e0) + gx) * _STRIDE)
            v1 = in_ref[pl.ds(_S + rv * 16, 16)]
            e1 = jnp.exp(v1)
            plsc.store_scatter(out_ref, [base_idx + 1],
                               (e1 / (1.0 + e1) + gy) * _STRIDE)
            v2 = in_ref[pl.ds(2 * _S + rv * 16, 16)]
            plsc.store_scatter(out_ref, [base_idx + 2], jnp.exp(v2) * aw)
            v3 = in_ref[pl.ds(3 * _S + rv * 16, 16)]
            plsc.store_scatter(out_ref, [base_idx + 3], jnp.exp(v3) * ah)

        out_base = ((wid * 3 + pi) * _NCHUNK + (t & (_NCHUNK - 1))) * _OUT_ITEM
        pltpu.async_copy(out_ref, out_hbm.at[pl.ds(out_base, _OUT_ITEM)], osem)

    fire(0, in_a, isem_a)
    fire(1, in_b, isem_b)

    def step(t2, carry):
        ta = t2 * 2
        tb = ta + 1

        @pl.when(t2 > 0)
        def _():
            drain_out(out_a, osem_a)

        wait_in(in_a, isem_a)
        compute(ta, in_a, out_a, osem_a)

        @pl.when(ta + 2 < _ITEMS_PER_W)
        def _():
            fire(ta + 2, in_a, isem_a)

        @pl.when(t2 > 0)
        def _():
            drain_out(out_b, osem_b)

        wait_in(in_b, isem_b)
        compute(tb, in_b, out_b, osem_b)

        @pl.when(tb + 2 < _ITEMS_PER_W)
        def _():
            fire(tb + 2, in_b, isem_b)

        return carry

    lax.fori_loop(0, _ITEMS_PER_W // 2, step, 0, unroll=False)
    drain_out(out_a, osem_a)
    drain_out(out_b, osem_b)


_sc_call = functools.partial(
    pl.kernel,
    mesh=plsc.VectorSubcoreMesh(core_axis_name="c", subcore_axis_name="s"),
    out_type=jax.ShapeDtypeStruct((_ITEMS * _OUT_ITEM,), jnp.float32),
    scratch_types=[
        pltpu.VMEM((_IN_ITEM,), jnp.float32),
        pltpu.VMEM((_IN_ITEM,), jnp.float32),
        pltpu.VMEM((_OUT_ITEM,), jnp.float32),
        pltpu.VMEM((_OUT_ITEM,), jnp.float32),
        pltpu.SemaphoreType.DMA,
        pltpu.SemaphoreType.DMA,
        pltpu.SemaphoreType.DMA,
        pltpu.SemaphoreType.DMA,
    ],
    compiler_params=pltpu.CompilerParams(
        needs_layout_passes=False, use_tc_tiling_on_sc=True),
)(_sc_body)


@jax.jit
def kernel(x):
    B = x.shape[0]
    x1 = x.reshape(B * 3 * _C * 64 * 64)
    out = _sc_call(x1)
    return out.reshape(B, 3 * 64 * 64, _C)


# speedup vs baseline: 10.2032x; 4.3183x over previous
import functools

import jax
import jax.numpy as jnp
from jax.experimental import pallas as pl

_ANCHOR_W = (10.0, 16.0, 33.0)
_ANCHOR_H = (13.0, 30.0, 23.0)
_STRIDE = 8.0
_C = 85
_G = 64
_POS = _G * _G
_B = 32
_CHUNK = 256
_K = _POS // _CHUNK


def _decode_body(x_ref, o_ref):
    a = pl.program_id(0)
    k = pl.program_id(1)
    v = x_ref[...]
    t = jnp.transpose(v, (2, 0, 1))
    ta = jnp.where(a == 0, t[0:_C],
         jnp.where(a == 1, t[_C:2 * _C], t[2 * _C:3 * _C]))
    e = jnp.exp(ta)
    sig = e / (1.0 + e)

    c_io = jax.lax.broadcasted_iota(jnp.int32, ta.shape, 0)
    g_io = jax.lax.broadcasted_iota(jnp.int32, ta.shape, 2)
    gx = (g_io & (_G - 1)).astype(jnp.float32)
    gy = (k * (_CHUNK // _G) + (g_io >> 6)).astype(jnp.float32)

    aw = jnp.where(a == 0, _ANCHOR_W[0],
                   jnp.where(a == 1, _ANCHOR_W[1], _ANCHOR_W[2]))
    ah = jnp.where(a == 0, _ANCHOR_H[0],
                   jnp.where(a == 1, _ANCHOR_H[1], _ANCHOR_H[2]))

    out = jnp.where(c_io == 0, (sig + gx) * _STRIDE,
          jnp.where(c_io == 1, (sig + gy) * _STRIDE,
          jnp.where(c_io == 2, e * aw,
          jnp.where(c_io == 3, e * ah, sig))))
    o_ref[...] = out


@jax.jit
def kernel(x):
    xt = jnp.transpose(x, (0, 2, 3, 1)).reshape(_B, _POS, 3 * _C)
    y = pl.pallas_call(
        _decode_body,
        grid=(3, _K),
        in_specs=[pl.BlockSpec((_B, _CHUNK, 3 * _C), lambda a, k: (0, k, 0))],
        out_specs=pl.BlockSpec((_C, _B, _CHUNK), lambda a, k: (0, 0, a * _K + k)),
        out_shape=jax.ShapeDtypeStruct((_C, _B, 3 * _POS), jnp.float32),
    )(xt)
    return jnp.transpose(y, (1, 2, 0))


# speedup vs baseline: 10.5890x; 1.0378x over previous
import functools

import jax
import jax.numpy as jnp
from jax.experimental import pallas as pl

_ANCHOR_W = (10.0, 16.0, 33.0)
_ANCHOR_H = (13.0, 30.0, 23.0)
_STRIDE = 8.0
_C = 85
_G = 64
_POS = _G * _G
_B = 32
_CHUNK = 256
_K = _POS // _CHUNK


def _decode_body(x_ref, o_ref):
    k = pl.program_id(0)
    a = pl.program_id(1)
    v = x_ref[...]
    t = jnp.transpose(v, (2, 0, 1))
    ta = jnp.where(a == 0, t[0:_C],
         jnp.where(a == 1, t[_C:2 * _C], t[2 * _C:3 * _C]))
    e = jnp.exp(ta)
    sig = e / (1.0 + e)

    c_io = jax.lax.broadcasted_iota(jnp.int32, ta.shape, 0)
    g_io = jax.lax.broadcasted_iota(jnp.int32, ta.shape, 2)
    gx = (g_io & (_G - 1)).astype(jnp.float32)
    gy = (k * (_CHUNK // _G) + (g_io >> 6)).astype(jnp.float32)

    aw = jnp.where(a == 0, _ANCHOR_W[0],
                   jnp.where(a == 1, _ANCHOR_W[1], _ANCHOR_W[2]))
    ah = jnp.where(a == 0, _ANCHOR_H[0],
                   jnp.where(a == 1, _ANCHOR_H[1], _ANCHOR_H[2]))

    out = jnp.where(c_io == 0, (sig + gx) * _STRIDE,
          jnp.where(c_io == 1, (sig + gy) * _STRIDE,
          jnp.where(c_io == 2, e * aw,
          jnp.where(c_io == 3, e * ah, sig))))
    o_ref[...] = out


@jax.jit
def kernel(x):
    xt = jnp.transpose(x, (0, 2, 3, 1)).reshape(_B, _POS, 3 * _C)
    y = pl.pallas_call(
        _decode_body,
        grid=(_K, 3),
        in_specs=[pl.BlockSpec((_B, _CHUNK, 3 * _C), lambda k, a: (0, k, 0))],
        out_specs=pl.BlockSpec((_C, _B, _CHUNK), lambda k, a: (0, 0, a * _K + k)),
        out_shape=jax.ShapeDtypeStruct((_C, _B, 3 * _POS), jnp.float32),
    )(xt)
    return jnp.transpose(y, (1, 2, 0))


# speedup vs baseline: 12.0266x; 1.1358x over previous
import functools

import jax
import jax.numpy as jnp
from jax.experimental import pallas as pl
from jax.experimental.pallas import tpu as pltpu

_ANCHOR_W = (10.0, 16.0, 33.0)
_ANCHOR_H = (13.0, 30.0, 23.0)
_STRIDE = 8.0
_C = 85
_G = 64
_POS = _G * _G
_B = 32
_CHUNK = 256
_K = _POS // _CHUNK


def _decode_body(x_ref, o_ref, scr_ref):
    k = pl.program_id(0)
    a = pl.program_id(1)

    @pl.when(a == 0)
    def _():
        scr_ref[...] = jnp.transpose(x_ref[...], (2, 0, 1))

    ta = scr_ref[pl.ds(a * _C, _C)]
    e = jnp.exp(ta)
    sig = e / (1.0 + e)

    e_h = e[0:8]
    sig_h = sig[0:8]
    c_io = jax.lax.broadcasted_iota(jnp.int32, e_h.shape, 0)
    g_io = jax.lax.broadcasted_iota(jnp.int32, e_h.shape, 2)
    gx = (g_io & (_G - 1)).astype(jnp.float32)
    gy = (k * (_CHUNK // _G) + (g_io >> 6)).astype(jnp.float32)

    aw = jnp.where(a == 0, _ANCHOR_W[0],
                   jnp.where(a == 1, _ANCHOR_W[1], _ANCHOR_W[2]))
    ah = jnp.where(a == 0, _ANCHOR_H[0],
                   jnp.where(a == 1, _ANCHOR_H[1], _ANCHOR_H[2]))

    head = jnp.where(c_io == 0, (sig_h + gx) * _STRIDE,
           jnp.where(c_io == 1, (sig_h + gy) * _STRIDE,
           jnp.where(c_io == 2, e_h * aw,
           jnp.where(c_io == 3, e_h * ah, sig_h))))
    o_ref[...] = jnp.concatenate([head, sig[8:]], axis=0)


@jax.jit
def kernel(x):
    xt = jnp.transpose(x, (0, 2, 3, 1)).reshape(_B, _POS, 3 * _C)
    y = pl.pallas_call(
        _decode_body,
        grid=(_K, 3),
        in_specs=[pl.BlockSpec((_B, _CHUNK, 3 * _C), lambda k, a: (0, k, 0))],
        out_specs=pl.BlockSpec((_C, _B, _CHUNK), lambda k, a: (0, 0, a * _K + k)),
        out_shape=jax.ShapeDtypeStruct((_C, _B, 3 * _POS), jnp.float32),
        scratch_shapes=[pltpu.VMEM((3 * _C, _B, _CHUNK), jnp.float32)],
    )(xt)
    return jnp.transpose(y, (1, 2, 0))
